# Initial kernel scaffold; baseline (speedup 1.0000x reference)
import jax
import jax.numpy as jnp
from jax.experimental import pallas as pl


def kernel(image_shape, boxes, scores, fpn0, fpn1, fpn2, fpn3, fpn4):
    raise NotImplementedError("write your pallas kernel here")


# speedup vs baseline: 16.4512x; 16.4512x over previous
import jax
import jax.numpy as jnp
import numpy as np
from jax import lax
from jax.experimental import pallas as pl
from jax.experimental.pallas import tpu as pltpu
from jax.experimental.pallas import tpu_sc as plsc

CH, CW = 14, 14
C = 256
IMG = 512.0
EPS = 1e-7
SIZES = (64, 32, 16, 8, 4)
BASES = (0, 4096, 5120, 5376, 5440)
NC, NS = 2, 16
NW = NC * NS
NPAD = 1024
BPW = NPAD // NW
THRESH = tuple((224.0 * (2.0 ** (k - 1) - EPS)) ** 2 for k in (1, 2, 3, 4))
RIMG = 1.0 / IMG
R13 = float(np.float32(1.0) / np.float32(13.0))
RHM1 = tuple(float(np.float32(1.0) / np.float32(s - 1)) for s in SIZES)


def _ifloor(v):
    t = v.astype(jnp.int32)
    return t - jnp.where(t.astype(jnp.float32) > v, jnp.int32(1), jnp.int32(0))


def _fdiv(a, b, rb):
    q = a * rb
    return q + (a - q * b) * rb


def _roi_body(table, boxes, out, boxes_v, idx_v, rows_v, outbuf, sem):
    wid = lax.axis_index("s") * NC + lax.axis_index("c")
    box0 = wid * BPW
    pltpu.sync_copy(boxes.at[pl.ds(box0 * 4, BPW * 4)], boxes_v)

    def quad_loop(q, carry):
        quad = boxes_v[pl.ds(q * 16, 16)]
        for r in range(4):
            _one_box(table, out, idx_v, rows_v, outbuf, sem,
                     box0 + q * 4 + r,
                     quad[4 * r + 0], quad[4 * r + 1],
                     quad[4 * r + 2], quad[4 * r + 3])
        return carry

    lax.fori_loop(0, BPW // 4, quad_loop, 0)


def _one_box(table, out, idx_v, rows_v, outbuf, sem, g, x1, y1, x2, y2):
    if True:
        wh = (x2 - x1) * (y2 - y1)
        lvl = ((wh >= THRESH[0]).astype(jnp.int32)
               + (wh >= THRESH[1]).astype(jnp.int32)
               + (wh >= THRESH[2]).astype(jnp.int32)
               + (wh >= THRESH[3]).astype(jnp.int32))
        fh = jnp.float32(SIZES[0])
        base = jnp.int32(BASES[0])
        rhm1 = jnp.float32(RHM1[0])
        for k in range(1, 5):
            sel = lvl == k
            fh = jnp.where(sel, jnp.float32(SIZES[k]), fh)
            base = jnp.where(sel, jnp.int32(BASES[k]), base)
            rhm1 = jnp.where(sel, jnp.float32(RHM1[k]), rhm1)
        w = fh.astype(jnp.int32)
        hm1 = fh - 1.0
        y1n = _fdiv(y1 * RIMG * fh, hm1, rhm1)
        x1n = _fdiv(x1 * RIMG * fh, hm1, rhm1)
        y2n = _fdiv(y2 * RIMG * fh - 1.0, hm1, rhm1)
        x2n = _fdiv(x2 * RIMG * fh - 1.0, hm1, rhm1)

        tx_v = _fdiv(lax.iota(jnp.int32, 16).astype(jnp.float32), 13.0, R13)
        xs_v = (x1n + (x2n - x1n) * tx_v) * hm1
        x0t_v = _ifloor(xs_v)
        x0c_v = jnp.minimum(jnp.maximum(x0t_v, 0), w - 1)
        x1c_v = jnp.minimum(jnp.maximum(x0t_v + 1, 0), w - 1)

        def row_loop(i, carry2):
            ty = _fdiv(i.astype(jnp.float32), 13.0, R13)
            ys = (y1n + (y2n - y1n) * ty) * hm1
            vy = jnp.where((ys >= 0.0) & (ys <= hm1), 1.0, 0.0)
            y0t = _ifloor(ys)
            ay = ys - y0t.astype(jnp.float32)
            y0c = jnp.minimum(jnp.maximum(y0t, 0), w - 1)
            y1c = jnp.minimum(jnp.maximum(y0t + 1, 0), w - 1)
            r0 = base + y0c * w
            r1 = base + y1c * w
            idx_v[pl.ds(0, 16)] = r0 + x0c_v
            idx_v[pl.ds(16, 16)] = r0 + x1c_v
            idx_v[pl.ds(32, 16)] = r1 + x0c_v
            idx_v[pl.ds(48, 16)] = r1 + x1c_v
            pltpu.async_copy(table.at[idx_v], rows_v, sem).wait()

            def col_loop(j, carry3):
                tx = _fdiv(j.astype(jnp.float32), 13.0, R13)
                xs = (x1n + (x2n - x1n) * tx) * hm1
                vx = jnp.where((xs >= 0.0) & (xs <= hm1), 1.0, 0.0)
                x0t = _ifloor(xs)
                ax = xs - x0t.astype(jnp.float32)
                m = vy * vx
                w00 = (1.0 - ay) * (1.0 - ax) * m
                w01 = (1.0 - ay) * ax * m
                w10 = ay * (1.0 - ax) * m
                w11 = ay * ax * m
                orow = i * 14 + j

                def ch_loop(cc, carry4):
                    cs = pl.ds(cc * 16, 16)
                    acc = (w00 * rows_v[j, cs] + w01 * rows_v[j + 16, cs]
                           + w10 * rows_v[j + 32, cs] + w11 * rows_v[j + 48, cs])
                    outbuf[orow, cs] = acc
                    return carry4

                return lax.fori_loop(0, C // 16, ch_loop, carry3)

            return lax.fori_loop(0, CW, col_loop, carry2)

        lax.fori_loop(0, CH, row_loop, 0)
        pltpu.sync_copy(outbuf, out.at[g])


@jax.jit
def _roi_align(table, boxes_p):
    mesh = plsc.VectorSubcoreMesh(core_axis_name="c", subcore_axis_name="s",
                                  num_cores=NC, num_subcores=NS)
    f = pl.kernel(
        _roi_body,
        out_type=jax.ShapeDtypeStruct((NPAD, CH * CW, C), jnp.float32),
        mesh=mesh,
        scratch_types=[
            pltpu.VMEM((BPW * 4,), jnp.float32),
            pltpu.VMEM((64,), jnp.int32),
            pltpu.VMEM((64, C), jnp.float32),
            pltpu.VMEM((CH * CW, C), jnp.float32),
            pltpu.SemaphoreType.DMA,
        ],
    )
    return f(table, boxes_p)


def kernel(image_shape, boxes, scores, fpn0, fpn1, fpn2, fpn3, fpn4):
    del image_shape, scores
    table = jnp.concatenate(
        [f[0].reshape(-1, C).astype(jnp.float32)
         for f in (fpn0, fpn1, fpn2, fpn3, fpn4)], axis=0)
    b0 = boxes[0].astype(jnp.float32)
    n = b0.shape[0]
    pad = jnp.broadcast_to(jnp.array([0.0, 0.0, 1.0, 1.0], jnp.float32),
                           (NPAD - n, 4))
    boxes_p = jnp.concatenate([b0, pad], axis=0).reshape(-1)
    out = _roi_align(table, boxes_p)
    return out[:n].reshape(1, n, CH, CW, C)


# speedup vs baseline: 18.3768x; 1.1171x over previous
import jax
import jax.numpy as jnp
import numpy as np
from jax import lax
from jax.experimental import pallas as pl
from jax.experimental.pallas import tpu as pltpu
from jax.experimental.pallas import tpu_sc as plsc

CH, CW = 14, 14
C = 256
IMG = 512.0
EPS = 1e-7
SIZES = (64, 32, 16, 8, 4)
BASES = (0, 4096, 5120, 5376, 5440)
NC, NS = 2, 16
NW = NC * NS
NPAD = 1024
BPW = NPAD // NW
THRESH = tuple((224.0 * (2.0 ** (k - 1) - EPS)) ** 2 for k in (1, 2, 3, 4))
RIMG = 1.0 / IMG
R13 = float(np.float32(1.0) / np.float32(13.0))
RHM1 = tuple(float(np.float32(1.0) / np.float32(s - 1)) for s in SIZES)
TY = tuple(float(np.float32(i) / np.float32(13.0)) for i in range(CH))


def _ifloor(v):
    t = v.astype(jnp.int32)
    return t - jnp.where(t.astype(jnp.float32) > v, jnp.int32(1), jnp.int32(0))


def _fdiv(a, b, rb):
    q = a * rb
    return q + (a - q * b) * rb


def _one_box(table, out, bufs, g, x1, y1, x2, y2):
    (idx2, rows2, orow2, semg2, semo2) = bufs
    wh = (x2 - x1) * (y2 - y1)
    lvl = ((wh >= THRESH[0]).astype(jnp.int32)
           + (wh >= THRESH[1]).astype(jnp.int32)
           + (wh >= THRESH[2]).astype(jnp.int32)
           + (wh >= THRESH[3]).astype(jnp.int32))
    fh = jnp.float32(SIZES[0])
    base = jnp.int32(BASES[0])
    rhm1 = jnp.float32(RHM1[0])
    for k in range(1, 5):
        sel = lvl == k
        fh = jnp.where(sel, jnp.float32(SIZES[k]), fh)
        base = jnp.where(sel, jnp.int32(BASES[k]), base)
        rhm1 = jnp.where(sel, jnp.float32(RHM1[k]), rhm1)
    w = fh.astype(jnp.int32)
    hm1 = fh - 1.0
    y1n = _fdiv(y1 * RIMG * fh, hm1, rhm1)
    x1n = _fdiv(x1 * RIMG * fh, hm1, rhm1)
    y2n = _fdiv(y2 * RIMG * fh - 1.0, hm1, rhm1)
    x2n = _fdiv(x2 * RIMG * fh - 1.0, hm1, rhm1)

    tx_v = _fdiv(lax.iota(jnp.int32, 16).astype(jnp.float32), 13.0, R13)
    xs_v = (x1n + (x2n - x1n) * tx_v) * hm1
    x0t_v = _ifloor(xs_v)
    x0c_v = jnp.minimum(jnp.maximum(x0t_v, 0), w - 1)
    x1c_v = jnp.minimum(jnp.maximum(x0t_v + 1, 0), w - 1)

    def yrow(i):
        ys = (y1n + (y2n - y1n) * TY[i]) * hm1
        vy = jnp.where((ys >= 0.0) & (ys <= hm1), 1.0, 0.0)
        y0t = _ifloor(ys)
        ay = ys - y0t.astype(jnp.float32)
        y0c = jnp.minimum(jnp.maximum(y0t, 0), w - 1)
        y1c = jnp.minimum(jnp.maximum(y0t + 1, 0), w - 1)
        return base + y0c * w, base + y1c * w, vy, ay

    def issue(i):
        r0, r1, _, _ = yrow(i)
        idxb = idx2[i % 2]
        idxb[pl.ds(0, 16)] = r0 + x0c_v
        idxb[pl.ds(16, 16)] = r0 + x1c_v
        idxb[pl.ds(32, 16)] = r1 + x0c_v
        idxb[pl.ds(48, 16)] = r1 + x1c_v
        return pltpu.async_copy(table.at[idxb], rows2[i % 2], semg2[i % 2])

    def blend(i):
        _, _, vy, ay = yrow(i)
        rowb = rows2[i % 2]
        ob = orow2[i % 2]

        def col_loop(j, c):
            tx = _fdiv(j.astype(jnp.float32), 13.0, R13)
            xs = (x1n + (x2n - x1n) * tx) * hm1
            vx = jnp.where((xs >= 0.0) & (xs <= hm1), 1.0, 0.0)
            x0t = _ifloor(xs)
            ax = xs - x0t.astype(jnp.float32)
            m = vy * vx
            w00 = (1.0 - ay) * (1.0 - ax) * m
            w01 = (1.0 - ay) * ax * m
            w10 = ay * (1.0 - ax) * m
            w11 = ay * ax * m
            for cc in range(C // 16):
                cs = pl.ds(cc * 16, 16)
                ob[j, cs] = (w00 * rowb[j, cs] + w01 * rowb[j + 16, cs]
                             + w10 * rowb[j + 32, cs] + w11 * rowb[j + 48, cs])
            return c

        lax.fori_loop(0, CW, col_loop, 0)

    gd = [None, None]
    od = [None, None]
    gd[0] = issue(0)
    for i in range(CH):
        if i + 1 < CH:
            gd[(i + 1) % 2] = issue(i + 1)
        gd[i % 2].wait()
        if od[i % 2] is not None:
            od[i % 2].wait()
        blend(i)
        od[i % 2] = pltpu.async_copy(orow2[i % 2], out.at[g, i], semo2[i % 2])
    od[0].wait()
    od[1].wait()


def _roi_body(table, boxes, out, boxes_v, idxa, idxb, rowsa, rowsb, orowa,
              orowb, semga, semgb, semoa, semob):
    wid = lax.axis_index("s") * NC + lax.axis_index("c")
    box0 = wid * BPW
    pltpu.sync_copy(boxes.at[pl.ds(box0 * 4, BPW * 4)],
                    boxes_v.at[pl.ds(0, BPW * 4)])
    bufs = ((idxa, idxb), (rowsa, rowsb), (orowa, orowb),
            (semga, semgb), (semoa, semob))

    def box_loop(b, carry):
        quad = boxes_v[pl.ds(b * 4, 16)]
        _one_box(table, out, bufs, box0 + b,
                 quad[0], quad[1], quad[2], quad[3])
        return carry

    lax.fori_loop(0, BPW, box_loop, 0)


@jax.jit
def _roi_align(table, boxes_p):
    mesh = plsc.VectorSubcoreMesh(core_axis_name="c", subcore_axis_name="s",
                                  num_cores=NC, num_subcores=NS)
    f = pl.kernel(
        _roi_body,
        out_type=jax.ShapeDtypeStruct((NPAD, CH, CW, C), jnp.float32),
        mesh=mesh,
        scratch_types=[
            pltpu.VMEM((BPW * 4 + 16,), jnp.float32),
            pltpu.VMEM((64,), jnp.int32),
            pltpu.VMEM((64,), jnp.int32),
            pltpu.VMEM((64, C), jnp.float32),
            pltpu.VMEM((64, C), jnp.float32),
            pltpu.VMEM((CW, C), jnp.float32),
            pltpu.VMEM((CW, C), jnp.float32),
            pltpu.SemaphoreType.DMA,
            pltpu.SemaphoreType.DMA,
            pltpu.SemaphoreType.DMA,
            pltpu.SemaphoreType.DMA,
        ],
    )
    return f(table, boxes_p)


def kernel(image_shape, boxes, scores, fpn0, fpn1, fpn2, fpn3, fpn4):
    del image_shape, scores
    table = jnp.concatenate(
        [f[0].reshape(-1, C).astype(jnp.float32)
         for f in (fpn0, fpn1, fpn2, fpn3, fpn4)], axis=0)
    b0 = boxes[0].astype(jnp.float32)
    n = b0.shape[0]
    pad = jnp.broadcast_to(jnp.array([0.0, 0.0, 1.0, 1.0], jnp.float32),
                           (NPAD - n, 4))
    boxes_p = jnp.concatenate([b0, pad], axis=0).reshape(-1)
    out = _roi_align(table, boxes_p)
    return out[:n].reshape(1, n, CH, CW, C)


# speedup vs baseline: 32.6986x; 1.7793x over previous
import jax
import jax.numpy as jnp
import numpy as np
from jax import lax
from jax.experimental import pallas as pl
from jax.experimental.pallas import tpu as pltpu
from jax.experimental.pallas import tpu_sc as plsc

CH, CW = 14, 14
C = 256
IMG = 512.0
EPS = 1e-7
SIZES = (64, 32, 16, 8, 4)
BASES = (0, 4096, 5120, 5376, 5440)
NC, NS = 2, 16
NW = NC * NS
N = 1000
NPAD = 1024
BPW = NPAD // NW
THRESH = tuple((224.0 * (2.0 ** (k - 1) - EPS)) ** 2 for k in (1, 2, 3, 4))
RIMG = 1.0 / IMG
R13 = float(np.float32(1.0) / np.float32(13.0))
RHM1 = tuple(float(np.float32(1.0) / np.float32(s - 1)) for s in SIZES)
TY = tuple(float(np.float32(i) / np.float32(13.0)) for i in range(CH))


def _ifloor(v):
    t = v.astype(jnp.int32)
    return t - jnp.where(t.astype(jnp.float32) > v, jnp.int32(1), jnp.int32(0))


def _fdiv(a, b, rb):
    q = a * rb
    return q + (a - q * b) * rb


def _one_box(table, out, bufs, g, x1, y1, x2, y2):
    (idx2, rows2, orow2, semg2, semo2) = bufs
    wh = (x2 - x1) * (y2 - y1)
    lvl = ((wh >= THRESH[0]).astype(jnp.int32)
           + (wh >= THRESH[1]).astype(jnp.int32)
           + (wh >= THRESH[2]).astype(jnp.int32)
           + (wh >= THRESH[3]).astype(jnp.int32))
    fh = jnp.float32(SIZES[0])
    base = jnp.int32(BASES[0])
    rhm1 = jnp.float32(RHM1[0])
    for k in range(1, 5):
        sel = lvl == k
        fh = jnp.where(sel, jnp.float32(SIZES[k]), fh)
        base = jnp.where(sel, jnp.int32(BASES[k]), base)
        rhm1 = jnp.where(sel, jnp.float32(RHM1[k]), rhm1)
    w = fh.astype(jnp.int32)
    hm1 = fh - 1.0
    y1n = _fdiv(y1 * RIMG * fh, hm1, rhm1)
    x1n = _fdiv(x1 * RIMG * fh, hm1, rhm1)
    y2n = _fdiv(y2 * RIMG * fh - 1.0, hm1, rhm1)
    x2n = _fdiv(x2 * RIMG * fh - 1.0, hm1, rhm1)

    tx_v = _fdiv(lax.iota(jnp.int32, 16).astype(jnp.float32), 13.0, R13)
    xs_v = (x1n + (x2n - x1n) * tx_v) * hm1
    x0t_v = _ifloor(xs_v)
    x0c_v = jnp.minimum(jnp.maximum(x0t_v, 0), w - 1)
    x1c_v = jnp.minimum(jnp.maximum(x0t_v + 1, 0), w - 1)

    def yrow(i):
        ys = (y1n + (y2n - y1n) * TY[i]) * hm1
        vy = jnp.where((ys >= 0.0) & (ys <= hm1), 1.0, 0.0)
        y0t = _ifloor(ys)
        ay = ys - y0t.astype(jnp.float32)
        y0c = jnp.minimum(jnp.maximum(y0t, 0), w - 1)
        y1c = jnp.minimum(jnp.maximum(y0t + 1, 0), w - 1)
        return base + y0c * w, base + y1c * w, vy, ay

    def issue(i):
        r0, r1, _, _ = yrow(i)
        idxb = idx2[i % 2]
        idxb[pl.ds(0, 16)] = r0 + x0c_v
        idxb[pl.ds(16, 16)] = r0 + x1c_v
        idxb[pl.ds(32, 16)] = r1 + x0c_v
        idxb[pl.ds(48, 16)] = r1 + x1c_v
        return pltpu.async_copy(table.at[idxb], rows2[i % 2], semg2[i % 2])

    def blend(i):
        _, _, vy, ay = yrow(i)
        rowb = rows2[i % 2]
        ob = orow2[i % 2]

        def col_loop(j, c):
            tx = _fdiv(j.astype(jnp.float32), 13.0, R13)
            xs = (x1n + (x2n - x1n) * tx) * hm1
            vx = jnp.where((xs >= 0.0) & (xs <= hm1), 1.0, 0.0)
            x0t = _ifloor(xs)
            ax = xs - x0t.astype(jnp.float32)
            m = vy * vx
            w00 = (1.0 - ay) * (1.0 - ax) * m
            w01 = (1.0 - ay) * ax * m
            w10 = ay * (1.0 - ax) * m
            w11 = ay * ax * m
            for cc in range(C // 16):
                cs = pl.ds(cc * 16, 16)
                ob[j, cs] = (w00 * rowb[j, cs] + w01 * rowb[j + 16, cs]
                             + w10 * rowb[j + 32, cs] + w11 * rowb[j + 48, cs])
            return c

        lax.fori_loop(0, CW, col_loop, 0)

    gd = [None, None]
    od = [None, None]
    gd[0] = issue(0)
    for i in range(CH):
        if i + 1 < CH:
            gd[(i + 1) % 2] = issue(i + 1)
        gd[i % 2].wait()
        if od[i % 2] is not None:
            od[i % 2].wait()
        blend(i)
        od[i % 2] = pltpu.async_copy(orow2[i % 2], out.at[g, i], semo2[i % 2])
    od[0].wait()
    od[1].wait()


def _roi_body(table, boxes, out, boxes_v, idxa, idxb, rowsa, rowsb, orowa,
              orowb, semga, semgb, semoa, semob):
    wid = lax.axis_index("s") * NC + lax.axis_index("c")
    box0 = wid * BPW
    pltpu.sync_copy(boxes.at[pl.ds(box0 * 4, BPW * 4)],
                    boxes_v.at[pl.ds(0, BPW * 4)])
    bufs = ((idxa, idxb), (rowsa, rowsb), (orowa, orowb),
            (semga, semgb), (semoa, semob))

    def box_loop(b, carry):
        quad = boxes_v[pl.ds(b * 4, 16)]
        g = box0 + b

        @pl.when(g < N)
        def _():
            _one_box(table, out, bufs, g, quad[0], quad[1], quad[2], quad[3])

        return carry

    lax.fori_loop(0, BPW, box_loop, 0)


@jax.jit
def _roi_align(table, boxes_p):
    mesh = plsc.VectorSubcoreMesh(core_axis_name="c", subcore_axis_name="s",
                                  num_cores=NC, num_subcores=NS)
    f = pl.kernel(
        _roi_body,
        out_type=jax.ShapeDtypeStruct((N, CH, CW, C), jnp.float32),
        mesh=mesh,
        scratch_types=[
            pltpu.VMEM((BPW * 4 + 16,), jnp.float32),
            pltpu.VMEM((64,), jnp.int32),
            pltpu.VMEM((64,), jnp.int32),
            pltpu.VMEM((64, C), jnp.float32),
            pltpu.VMEM((64, C), jnp.float32),
            pltpu.VMEM((CW, C), jnp.float32),
            pltpu.VMEM((CW, C), jnp.float32),
            pltpu.SemaphoreType.DMA,
            pltpu.SemaphoreType.DMA,
            pltpu.SemaphoreType.DMA,
            pltpu.SemaphoreType.DMA,
        ],
    )
    return f(table, boxes_p)


def kernel(image_shape, boxes, scores, fpn0, fpn1, fpn2, fpn3, fpn4):
    del image_shape, scores
    table = jnp.concatenate(
        [f[0].reshape(-1, C).astype(jnp.float32)
         for f in (fpn0, fpn1, fpn2, fpn3, fpn4)], axis=0)
    b0 = boxes[0].astype(jnp.float32)
    n = b0.shape[0]
    pad = jnp.broadcast_to(jnp.array([0.0, 0.0, 1.0, 1.0], jnp.float32),
                           (NPAD - n, 4))
    boxes_p = jnp.concatenate([b0, pad], axis=0).reshape(-1)
    out = _roi_align(table, boxes_p)
    return out.reshape(1, n, CH, CW, C)


# speedup vs baseline: 32.7407x; 1.0013x over previous
import jax
import jax.numpy as jnp
import numpy as np
from jax import lax
from jax.experimental import pallas as pl
from jax.experimental.pallas import tpu as pltpu
from jax.experimental.pallas import tpu_sc as plsc

CH, CW = 14, 14
C = 256
IMG = 512.0
EPS = 1e-7
SIZES = (64, 32, 16, 8, 4)
BASES = (0, 4096, 5120, 5376, 5440)
NC, NS = 2, 16
NW = NC * NS
N = 1000
NPAD = 1024
BPW = NPAD // NW
THRESH = tuple((224.0 * (2.0 ** (k - 1) - EPS)) ** 2 for k in (1, 2, 3, 4))
RIMG = 1.0 / IMG
R13 = float(np.float32(1.0) / np.float32(13.0))
RHM1 = tuple(float(np.float32(1.0) / np.float32(s - 1)) for s in SIZES)
TY = tuple(float(np.float32(i) / np.float32(13.0)) for i in range(CH))


def _ifloor(v):
    t = v.astype(jnp.int32)
    return t - jnp.where(t.astype(jnp.float32) > v, jnp.int32(1), jnp.int32(0))


def _fdiv(a, b, rb):
    q = a * rb
    return q + (a - q * b) * rb


def _one_box(table, out, bufs, g, x1, y1, x2, y2):
    (idx2, rows2, orow2, semg2, semo2) = bufs
    wh = (x2 - x1) * (y2 - y1)
    lvl = ((wh >= THRESH[0]).astype(jnp.int32)
           + (wh >= THRESH[1]).astype(jnp.int32)
           + (wh >= THRESH[2]).astype(jnp.int32)
           + (wh >= THRESH[3]).astype(jnp.int32))
    fh = jnp.float32(SIZES[0])
    base = jnp.int32(BASES[0])
    rhm1 = jnp.float32(RHM1[0])
    for k in range(1, 5):
        sel = lvl == k
        fh = jnp.where(sel, jnp.float32(SIZES[k]), fh)
        base = jnp.where(sel, jnp.int32(BASES[k]), base)
        rhm1 = jnp.where(sel, jnp.float32(RHM1[k]), rhm1)
    w = fh.astype(jnp.int32)
    hm1 = fh - 1.0
    y1n = _fdiv(y1 * RIMG * fh, hm1, rhm1)
    x1n = _fdiv(x1 * RIMG * fh, hm1, rhm1)
    y2n = _fdiv(y2 * RIMG * fh - 1.0, hm1, rhm1)
    x2n = _fdiv(x2 * RIMG * fh - 1.0, hm1, rhm1)

    tx_v = _fdiv(lax.iota(jnp.int32, 16).astype(jnp.float32), 13.0, R13)
    xs_v = (x1n + (x2n - x1n) * tx_v) * hm1
    x0t_v = _ifloor(xs_v)
    x0c_v = jnp.minimum(jnp.maximum(x0t_v, 0), w - 1)
    x1c_v = jnp.minimum(jnp.maximum(x0t_v + 1, 0), w - 1)

    def yrow(i):
        ys = (y1n + (y2n - y1n) * TY[i]) * hm1
        vy = jnp.where((ys >= 0.0) & (ys <= hm1), 1.0, 0.0)
        y0t = _ifloor(ys)
        ay = ys - y0t.astype(jnp.float32)
        y0c = jnp.minimum(jnp.maximum(y0t, 0), w - 1)
        y1c = jnp.minimum(jnp.maximum(y0t + 1, 0), w - 1)
        return base + y0c * w, base + y1c * w, vy, ay

    def issue(i):
        r0, r1, _, _ = yrow(i)
        idxb = idx2[i % 2]
        idxb[pl.ds(0, 16)] = r0 + x0c_v
        idxb[pl.ds(16, 16)] = r0 + x1c_v
        idxb[pl.ds(32, 16)] = r1 + x0c_v
        idxb[pl.ds(48, 16)] = r1 + x1c_v
        return pltpu.async_copy(table.at[idxb], rows2[i % 2], semg2[i % 2])

    def blend(i):
        _, _, vy, ay = yrow(i)
        rowb = rows2[i % 2]
        ob = orow2[i % 2]

        def col_loop(j, c):
            tx = _fdiv(j.astype(jnp.float32), 13.0, R13)
            xs = (x1n + (x2n - x1n) * tx) * hm1
            vx = jnp.where((xs >= 0.0) & (xs <= hm1), 1.0, 0.0)
            x0t = _ifloor(xs)
            ax = xs - x0t.astype(jnp.float32)
            m = vy * vx
            w00 = (1.0 - ay) * (1.0 - ax) * m
            w01 = (1.0 - ay) * ax * m
            w10 = ay * (1.0 - ax) * m
            w11 = ay * ax * m
            for cc in range(C // 16):
                cs = pl.ds(cc * 16, 16)
                ob[j, cs] = (w00 * rowb[j, cs] + w01 * rowb[j + 16, cs]
                             + w10 * rowb[j + 32, cs] + w11 * rowb[j + 48, cs])
            return c

        lax.fori_loop(0, CW, col_loop, 0)

    gd = [None, None]
    od = [None, None]
    gd[0] = issue(0)
    for i in range(CH):
        if i + 1 < CH:
            gd[(i + 1) % 2] = issue(i + 1)
        gd[i % 2].wait()
        if od[i % 2] is not None:
            od[i % 2].wait()
        blend(i)
        od[i % 2] = pltpu.async_copy(orow2[i % 2], out.at[g, i], semo2[i % 2])
    od[0].wait()
    od[1].wait()


def _roi_body(table, boxes, out, boxes_v, idxa, idxb, rowsa, rowsb, orowa,
              orowb, semga, semgb, semoa, semob):
    wid = lax.axis_index("s") * NC + lax.axis_index("c")
    box0 = wid * BPW
    pltpu.sync_copy(boxes.at[pl.ds(box0 * 4, BPW * 4)],
                    boxes_v.at[pl.ds(0, BPW * 4)])
    bufs = ((idxa, idxb), (rowsa, rowsb), (orowa, orowb),
            (semga, semgb), (semoa, semob))

    def box_loop(b, carry):
        quad = boxes_v[pl.ds(b * 4, 16)]
        g = box0 + b

        @pl.when(g < N)
        def _():
            _one_box(table, out, bufs, g, quad[0], quad[1], quad[2], quad[3])

        return carry

    lax.fori_loop(0, BPW, box_loop, 0)


@jax.jit
def _roi_align(table, boxes_p):
    mesh = plsc.VectorSubcoreMesh(core_axis_name="c", subcore_axis_name="s",
                                  num_cores=NC, num_subcores=NS)
    f = pl.kernel(
        _roi_body,
        out_type=jax.ShapeDtypeStruct((N, CH, CW, C), jnp.float32),
        mesh=mesh,
        compiler_params=pltpu.CompilerParams(use_tc_tiling_on_sc=True),
        scratch_types=[
            pltpu.VMEM((BPW * 4 + 16,), jnp.float32),
            pltpu.VMEM((64,), jnp.int32),
            pltpu.VMEM((64,), jnp.int32),
            pltpu.VMEM((64, C), jnp.float32),
            pltpu.VMEM((64, C), jnp.float32),
            pltpu.VMEM((CW, C), jnp.float32),
            pltpu.VMEM((CW, C), jnp.float32),
            pltpu.SemaphoreType.DMA,
            pltpu.SemaphoreType.DMA,
            pltpu.SemaphoreType.DMA,
            pltpu.SemaphoreType.DMA,
        ],
    )
    return f(table, boxes_p)


def kernel(image_shape, boxes, scores, fpn0, fpn1, fpn2, fpn3, fpn4):
    del image_shape, scores
    table = jnp.concatenate(
        [f[0].reshape(-1, C).astype(jnp.float32)
         for f in (fpn0, fpn1, fpn2, fpn3, fpn4)], axis=0)
    b0 = boxes[0].astype(jnp.float32)
    n = b0.shape[0]
    pad = jnp.broadcast_to(jnp.array([0.0, 0.0, 1.0, 1.0], jnp.float32),
                           (NPAD - n, 4))
    boxes_p = jnp.concatenate([b0, pad], axis=0).reshape(-1)
    out = _roi_align(table, boxes_p)
    return out.reshape(1, n, CH, CW, C)


# speedup vs baseline: 49.4794x; 1.5113x over previous
import jax
import jax.numpy as jnp
import numpy as np
from jax import lax
from jax.experimental import pallas as pl
from jax.experimental.pallas import tpu as pltpu
from jax.experimental.pallas import tpu_sc as plsc

CH, CW = 14, 14
C = 256
IMG = 512.0
EPS = 1e-7
SIZES = (64, 32, 16, 8, 4)
BASES = (0, 4096, 5120, 5376, 5440)
NC, NS = 2, 16
NW = NC * NS
N = 1000
NPAD = 1024
BPW = NPAD // NW
GRP = 8
THRESH = tuple((224.0 * (2.0 ** (k - 1) - EPS)) ** 2 for k in (1, 2, 3, 4))
RIMG = 1.0 / IMG
R13 = float(np.float32(1.0) / np.float32(13.0))
RHM1 = tuple(float(np.float32(1.0) / np.float32(s - 1)) for s in SIZES)


def _ifloor(v):
    t = v.astype(jnp.int32)
    return t - jnp.where(t.astype(jnp.float32) > v, jnp.int32(1), jnp.int32(0))


def _fdiv(a, b, rb):
    q = a * rb
    return q + (a - q * b) * rb


def _pack16f(vals):
    lanes = lax.iota(jnp.int32, 16)
    v = jnp.full((16,), 0.0, jnp.float32)
    for k, s in enumerate(vals):
        v = jnp.where(lanes == k, s, v)
    return v


def _box_state(x1, y1, x2, y2):
    wh = (x2 - x1) * (y2 - y1)
    lvl = ((wh >= THRESH[0]).astype(jnp.int32)
           + (wh >= THRESH[1]).astype(jnp.int32)
           + (wh >= THRESH[2]).astype(jnp.int32)
           + (wh >= THRESH[3]).astype(jnp.int32))
    fh = jnp.float32(SIZES[0])
    base = jnp.int32(BASES[0])
    rhm1 = jnp.float32(RHM1[0])
    for k in range(1, 5):
        sel = lvl == k
        fh = jnp.where(sel, jnp.float32(SIZES[k]), fh)
        base = jnp.where(sel, jnp.int32(BASES[k]), base)
        rhm1 = jnp.where(sel, jnp.float32(RHM1[k]), rhm1)
    w = fh.astype(jnp.int32)
    hm1 = fh - 1.0
    y1n = _fdiv(y1 * RIMG * fh, hm1, rhm1)
    x1n = _fdiv(x1 * RIMG * fh, hm1, rhm1)
    y2n = _fdiv(y2 * RIMG * fh - 1.0, hm1, rhm1)
    x2n = _fdiv(x2 * RIMG * fh - 1.0, hm1, rhm1)
    tx_v = _fdiv(lax.iota(jnp.int32, 16).astype(jnp.float32), 13.0, R13)
    xs_v = (x1n + (x2n - x1n) * tx_v) * hm1
    x0t_v = _ifloor(xs_v)
    x0c_v = jnp.minimum(jnp.maximum(x0t_v, 0), w - 1)
    x1c_v = jnp.minimum(jnp.maximum(x0t_v + 1, 0), w - 1)
    packed = _pack16f([hm1, y1n, y2n, x1n, x2n,
                       base.astype(jnp.float32), fh])
    return packed, x0c_v, x1c_v


def _yrow(st, i):
    (base, w, hm1, y1n, y2n, _, _) = st
    ty = _fdiv(lax.convert_element_type(i, jnp.float32), 13.0, R13)
    ys = (y1n + (y2n - y1n) * ty) * hm1
    vy = jnp.where((ys >= 0.0) & (ys <= hm1), 1.0, 0.0)
    y0t = _ifloor(ys)
    ay = ys - y0t.astype(jnp.float32)
    y0c = jnp.minimum(jnp.maximum(y0t, 0), w - 1)
    y1c = jnp.minimum(jnp.maximum(y0t + 1, 0), w - 1)
    return base + y0c * w, base + y1c * w, vy, ay


def _load_state(stf, stx0, stx1, b8):
    vf = stf[b8, :]
    base = vf[5].astype(jnp.int32)
    w = vf[6].astype(jnp.int32)
    st = (base, w, vf[0], vf[1], vf[2], vf[3], vf[4])
    return st, stx0[b8, :], stx1[b8, :]


def _roi_body(table, boxes, out, boxes_v, stf, stx0, stx1, idxa, idxb,
              rowsa, rowsb, oba, obb, semga, semgb, semoa, semob):
    wid = lax.axis_index("s") * NC + lax.axis_index("c")
    box0 = wid * BPW
    pltpu.sync_copy(boxes.at[pl.ds(box0 * 4, BPW * 4)],
                    boxes_v.at[pl.ds(0, BPW * 4)])
    idx2 = (idxa, idxb)
    rows2 = (rowsa, rowsb)
    ob2 = (oba, obb)
    semg2 = (semga, semgb)
    semo2 = (semoa, semob)

    def grp_loop(grp, carry):
        gbase = box0 + grp * GRP

        @pl.when(gbase < N)
        def _():
            qa = boxes_v[pl.ds(grp * 32, 16)]
            qb = boxes_v[pl.ds(grp * 32 + 16, 16)]
            for b8 in range(GRP):
                q = qa if b8 < 4 else qb
                r = 4 * (b8 % 4)
                packed, x0c_v, x1c_v = _box_state(q[r + 0], q[r + 1],
                                                  q[r + 2], q[r + 3])
                stf[b8, :] = packed
                stx0[b8, :] = x0c_v
                stx1[b8, :] = x1c_v

            def issue(i, b8, par):
                st, x0c_v, x1c_v = _load_state(stf, stx0, stx1, b8)
                r0, r1, _, _ = _yrow(st, i)
                idxb_ = idx2[par]
                idxb_[pl.ds(0, 16)] = r0 + x0c_v
                idxb_[pl.ds(16, 16)] = r0 + x1c_v
                idxb_[pl.ds(32, 16)] = r1 + x0c_v
                idxb_[pl.ds(48, 16)] = r1 + x1c_v
                pltpu.async_copy(table.at[idxb_], rows2[par], semg2[par])

            def blend(i, b8, par, ob):
                st, _, _ = _load_state(stf, stx0, stx1, b8)
                (_, _, hm1, _, _, x1n, x2n) = st
                _, _, vy, ay = _yrow(st, i)
                rowb = rows2[par]

                def col_loop(j, c):
                    tx = _fdiv(j.astype(jnp.float32), 13.0, R13)
                    xs = (x1n + (x2n - x1n) * tx) * hm1
                    vx = jnp.where((xs >= 0.0) & (xs <= hm1), 1.0, 0.0)
                    x0t = _ifloor(xs)
                    ax = xs - x0t.astype(jnp.float32)
                    m = vy * vx
                    w00 = (1.0 - ay) * (1.0 - ax) * m
                    w01 = (1.0 - ay) * ax * m
                    w10 = ay * (1.0 - ax) * m
                    w11 = ay * ax * m
                    def ch_loop(cq, c2):
                        for u in range(4):
                            cs = pl.ds(cq * 64 + u * 16, 16)
                            ob[j, b8, cs] = (
                                w00 * rowb[j, cs] + w01 * rowb[j + 16, cs]
                                + w10 * rowb[j + 32, cs]
                                + w11 * rowb[j + 48, cs])
                        return c2

                    lax.fori_loop(0, 4, ch_loop, 0)
                    return c

                lax.fori_loop(0, CW, col_loop, 0)

            def slot(i, b8, par, obpar):
                nwrap = b8 == GRP - 1
                ni = jnp.where(nwrap, i + 1, i)
                nb8 = jnp.where(nwrap, 0, b8 + 1)

                @pl.when(ni < CH)
                def _():
                    issue(ni, nb8, 1 - par)

                pltpu.make_async_copy(table.at[idx2[par]], rows2[par],
                                      semg2[par]).wait()
                blend(i, b8, par, ob2[obpar])

            def half(i, obpar, t):
                @pl.when(t >= 1)
                def _():
                    pltpu.make_async_copy(
                        ob2[obpar],
                        out.at[0, pl.ds(0, CW), pl.ds(gbase, GRP)],
                        semo2[obpar]).wait()

                def b_loop(it, c2):
                    slot(i, it * 2, 0, obpar)
                    slot(i, it * 2 + 1, 1, obpar)
                    return c2

                lax.fori_loop(0, GRP // 2, b_loop, 0)
                pltpu.async_copy(ob2[obpar],
                                 out.at[i, pl.ds(0, CW), pl.ds(gbase, GRP)],
                                 semo2[obpar])

            issue(jnp.int32(0), jnp.int32(0), 0)

            def i_loop(t, carry2):
                half(2 * t, 0, t)
                half(2 * t + 1, 1, t)
                return carry2

            lax.fori_loop(0, CH // 2, i_loop, 0)
            for par in range(2):
                pltpu.make_async_copy(
                    ob2[par], out.at[0, pl.ds(0, CW), pl.ds(gbase, GRP)],
                    semo2[par]).wait()

        return carry

    lax.fori_loop(0, BPW // GRP, grp_loop, 0)


@jax.jit
def _roi_align(table, boxes_p):
    mesh = plsc.VectorSubcoreMesh(core_axis_name="c", subcore_axis_name="s",
                                  num_cores=NC, num_subcores=NS)
    f = pl.kernel(
        _roi_body,
        out_type=jax.ShapeDtypeStruct((CH, CW, N, C), jnp.float32),
        mesh=mesh,
        scratch_types=[
            pltpu.VMEM((BPW * 4 + 16,), jnp.float32),
            pltpu.VMEM((GRP, 16), jnp.float32),
            pltpu.VMEM((GRP, 16), jnp.int32),
            pltpu.VMEM((GRP, 16), jnp.int32),
            pltpu.VMEM((64,), jnp.int32),
            pltpu.VMEM((64,), jnp.int32),
            pltpu.VMEM((64, C), jnp.float32),
            pltpu.VMEM((64, C), jnp.float32),
            pltpu.VMEM((CW, GRP, C), jnp.float32),
            pltpu.VMEM((CW, GRP, C), jnp.float32),
            pltpu.SemaphoreType.DMA,
            pltpu.SemaphoreType.DMA,
            pltpu.SemaphoreType.DMA,
            pltpu.SemaphoreType.DMA,
        ],
    )
    return f(table, boxes_p)


def kernel(image_shape, boxes, scores, fpn0, fpn1, fpn2, fpn3, fpn4):
    del image_shape, scores
    table = jnp.concatenate(
        [f[0].reshape(-1, C).astype(jnp.float32)
         for f in (fpn0, fpn1, fpn2, fpn3, fpn4)], axis=0)
    b0 = boxes[0].astype(jnp.float32)
    n = b0.shape[0]
    pad = jnp.broadcast_to(jnp.array([0.0, 0.0, 1.0, 1.0], jnp.float32),
                           (NPAD - n, 4))
    boxes_p = jnp.concatenate([b0, pad], axis=0).reshape(-1)
    out = _roi_align(table, boxes_p)
    return jnp.transpose(out, (2, 0, 1, 3)).reshape(1, n, CH, CW, C)


# speedup vs baseline: 49.7752x; 1.0060x over previous
import jax
import jax.numpy as jnp
import numpy as np
from jax import lax
from jax.experimental import pallas as pl
from jax.experimental.pallas import tpu as pltpu
from jax.experimental.pallas import tpu_sc as plsc

CH, CW = 14, 14
C = 256
IMG = 512.0
EPS = 1e-7
SIZES = (64, 32, 16, 8, 4)
BASES = (0, 4096, 5120, 5376, 5440)
NC, NS = 2, 16
NW = NC * NS
N = 1000
NPAD = 1024
BPW = NPAD // NW
GRP = 8
THRESH = tuple((224.0 * (2.0 ** (k - 1) - EPS)) ** 2 for k in (1, 2, 3, 4))
RIMG = 1.0 / IMG
R13 = float(np.float32(1.0) / np.float32(13.0))
RHM1 = tuple(float(np.float32(1.0) / np.float32(s - 1)) for s in SIZES)


def _ifloor(v):
    t = v.astype(jnp.int32)
    return t - jnp.where(t.astype(jnp.float32) > v, jnp.int32(1), jnp.int32(0))


def _fdiv(a, b, rb):
    q = a * rb
    return q + (a - q * b) * rb


_GDN = lax.GatherDimensionNumbers(offset_dims=(), collapsed_slice_dims=(0,),
                                  start_index_map=(0,))


def _vgather(v, idx):
    return lax.gather(v, idx[:, None], _GDN, slice_sizes=(1,),
                      mode=lax.GatherScatterMode.PROMISE_IN_BOUNDS)


def _pack16f(vals):
    lanes = lax.iota(jnp.int32, 16)
    v = jnp.full((16,), 0.0, jnp.float32)
    for k, s in enumerate(vals):
        v = jnp.where(lanes == k, s, v)
    return v


def _box_state(x1, y1, x2, y2):
    wh = (x2 - x1) * (y2 - y1)
    lvl = ((wh >= THRESH[0]).astype(jnp.int32)
           + (wh >= THRESH[1]).astype(jnp.int32)
           + (wh >= THRESH[2]).astype(jnp.int32)
           + (wh >= THRESH[3]).astype(jnp.int32))
    fh = jnp.float32(SIZES[0])
    base = jnp.int32(BASES[0])
    rhm1 = jnp.float32(RHM1[0])
    for k in range(1, 5):
        sel = lvl == k
        fh = jnp.where(sel, jnp.float32(SIZES[k]), fh)
        base = jnp.where(sel, jnp.int32(BASES[k]), base)
        rhm1 = jnp.where(sel, jnp.float32(RHM1[k]), rhm1)
    w = fh.astype(jnp.int32)
    hm1 = fh - 1.0
    y1n = _fdiv(y1 * RIMG * fh, hm1, rhm1)
    x1n = _fdiv(x1 * RIMG * fh, hm1, rhm1)
    y2n = _fdiv(y2 * RIMG * fh - 1.0, hm1, rhm1)
    x2n = _fdiv(x2 * RIMG * fh - 1.0, hm1, rhm1)
    tx_v = _fdiv(lax.iota(jnp.int32, 16).astype(jnp.float32), 13.0, R13)
    xs_v = (x1n + (x2n - x1n) * tx_v) * hm1
    x0t_v = _ifloor(xs_v)
    x0c_v = jnp.minimum(jnp.maximum(x0t_v, 0), w - 1)
    x1c_v = jnp.minimum(jnp.maximum(x0t_v + 1, 0), w - 1)
    packed = _pack16f([hm1, y1n, y2n, x1n, x2n,
                       base.astype(jnp.float32), fh])
    return packed, x0c_v, x1c_v


def _yrow(st, i):
    (base, w, hm1, y1n, y2n, _, _) = st
    ty = _fdiv(lax.convert_element_type(i, jnp.float32), 13.0, R13)
    ys = (y1n + (y2n - y1n) * ty) * hm1
    vy = jnp.where((ys >= 0.0) & (ys <= hm1), 1.0, 0.0)
    y0t = _ifloor(ys)
    ay = ys - y0t.astype(jnp.float32)
    y0c = jnp.minimum(jnp.maximum(y0t, 0), w - 1)
    y1c = jnp.minimum(jnp.maximum(y0t + 1, 0), w - 1)
    return base + y0c * w, base + y1c * w, vy, ay


def _load_state(stf, stx0, stx1, b8):
    vf = stf[b8, :]
    base = vf[5].astype(jnp.int32)
    w = vf[6].astype(jnp.int32)
    st = (base, w, vf[0], vf[1], vf[2], vf[3], vf[4])
    return st, stx0[b8, :], stx1[b8, :]


def _roi_body(table, boxes, out, boxes_v, stf, stx0, stx1, idxa, idxb,
              rowsa, rowsb, oba, obb, semga, semgb, semoa, semob):
    wid = lax.axis_index("s") * NC + lax.axis_index("c")
    box0 = wid * BPW
    pltpu.sync_copy(boxes.at[pl.ds(box0 * 4, BPW * 4)],
                    boxes_v.at[pl.ds(0, BPW * 4)])
    idx2 = (idxa, idxb)
    rows2 = (rowsa, rowsb)
    ob2 = (oba, obb)
    semg2 = (semga, semgb)
    semo2 = (semoa, semob)

    def grp_loop(grp, carry):
        gbase = box0 + grp * GRP

        @pl.when(gbase < N)
        def _():
            qa = boxes_v[pl.ds(grp * 32, 16)]
            qb = boxes_v[pl.ds(grp * 32 + 16, 16)]
            for b8 in range(GRP):
                q = qa if b8 < 4 else qb
                r = 4 * (b8 % 4)
                packed, x0c_v, x1c_v = _box_state(q[r + 0], q[r + 1],
                                                  q[r + 2], q[r + 3])
                stf[b8, :] = packed
                stx0[b8, :] = x0c_v
                stx1[b8, :] = x1c_v

            lanes = lax.iota(jnp.int32, 16)

            def issue(i, b8, par):
                st, x0c_v, x1c_v = _load_state(stf, stx0, stx1, b8)
                r0, r1, _, _ = _yrow(st, i)
                a = r0 + x0c_v
                b = r0 + x1c_v
                c = r1 + x0c_v
                d = r1 + x1c_v
                cl = lambda k: jnp.minimum(jnp.maximum(lanes + k, 0), 15)
                v0 = jnp.where(lanes < 14, a, _vgather(b, cl(-14)))
                v1 = jnp.where(lanes < 12, _vgather(b, cl(2)),
                               _vgather(c, cl(-12)))
                v2 = jnp.where(lanes < 10, _vgather(c, cl(4)),
                               _vgather(d, cl(-10)))
                v3 = _vgather(d, jnp.minimum(lanes + 6, 13))
                idxb_ = idx2[par]
                idxb_[pl.ds(0, 16)] = v0
                idxb_[pl.ds(16, 16)] = v1
                idxb_[pl.ds(32, 16)] = v2
                idxb_[pl.ds(48, 16)] = v3
                pltpu.async_copy(table.at[idxb_.at[pl.ds(0, 56)]],
                                 rows2[par], semg2[par])

            def blend(i, b8, par, ob):
                st, _, _ = _load_state(stf, stx0, stx1, b8)
                (_, _, hm1, _, _, x1n, x2n) = st
                _, _, vy, ay = _yrow(st, i)
                rowb = rows2[par]

                def col_loop(j, c):
                    tx = _fdiv(j.astype(jnp.float32), 13.0, R13)
                    xs = (x1n + (x2n - x1n) * tx) * hm1
                    vx = jnp.where((xs >= 0.0) & (xs <= hm1), 1.0, 0.0)
                    x0t = _ifloor(xs)
                    ax = xs - x0t.astype(jnp.float32)
                    m = vy * vx
                    w00 = (1.0 - ay) * (1.0 - ax) * m
                    w01 = (1.0 - ay) * ax * m
                    w10 = ay * (1.0 - ax) * m
                    w11 = ay * ax * m
                    def ch_loop(cq, c2):
                        for u in range(4):
                            cs = pl.ds(cq * 64 + u * 16, 16)
                            ob[j, b8, cs] = (
                                w00 * rowb[j, cs] + w01 * rowb[j + 14, cs]
                                + w10 * rowb[j + 28, cs]
                                + w11 * rowb[j + 42, cs])
                        return c2

                    lax.fori_loop(0, 4, ch_loop, 0)
                    return c

                lax.fori_loop(0, CW, col_loop, 0)

            def slot(i, b8, par, obpar):
                nwrap = b8 == GRP - 1
                ni = jnp.where(nwrap, i + 1, i)
                nb8 = jnp.where(nwrap, 0, b8 + 1)

                @pl.when(ni < CH)
                def _():
                    issue(ni, nb8, 1 - par)

                pltpu.make_async_copy(table.at[idx2[par].at[pl.ds(0, 56)]],
                                      rows2[par], semg2[par]).wait()
                blend(i, b8, par, ob2[obpar])

            def half(i, obpar, t):
                @pl.when(t >= 1)
                def _():
                    pltpu.make_async_copy(
                        ob2[obpar],
                        out.at[0, pl.ds(0, CW), pl.ds(gbase, GRP)],
                        semo2[obpar]).wait()

                def b_loop(it, c2):
                    slot(i, it * 2, 0, obpar)
                    slot(i, it * 2 + 1, 1, obpar)
                    return c2

                lax.fori_loop(0, GRP // 2, b_loop, 0)
                pltpu.async_copy(ob2[obpar],
                                 out.at[i, pl.ds(0, CW), pl.ds(gbase, GRP)],
                                 semo2[obpar])

            issue(jnp.int32(0), jnp.int32(0), 0)

            def i_loop(t, carry2):
                half(2 * t, 0, t)
                half(2 * t + 1, 1, t)
                return carry2

            lax.fori_loop(0, CH // 2, i_loop, 0)
            for par in range(2):
                pltpu.make_async_copy(
                    ob2[par], out.at[0, pl.ds(0, CW), pl.ds(gbase, GRP)],
                    semo2[par]).wait()

        return carry

    lax.fori_loop(0, BPW // GRP, grp_loop, 0)


@jax.jit
def _roi_align(table, boxes_p):
    mesh = plsc.VectorSubcoreMesh(core_axis_name="c", subcore_axis_name="s",
                                  num_cores=NC, num_subcores=NS)
    f = pl.kernel(
        _roi_body,
        out_type=jax.ShapeDtypeStruct((CH, CW, N, C), jnp.float32),
        mesh=mesh,
        scratch_types=[
            pltpu.VMEM((BPW * 4 + 16,), jnp.float32),
            pltpu.VMEM((GRP, 16), jnp.float32),
            pltpu.VMEM((GRP, 16), jnp.int32),
            pltpu.VMEM((GRP, 16), jnp.int32),
            pltpu.VMEM((64,), jnp.int32),
            pltpu.VMEM((64,), jnp.int32),
            pltpu.VMEM((56, C), jnp.float32),
            pltpu.VMEM((56, C), jnp.float32),
            pltpu.VMEM((CW, GRP, C), jnp.float32),
            pltpu.VMEM((CW, GRP, C), jnp.float32),
            pltpu.SemaphoreType.DMA,
            pltpu.SemaphoreType.DMA,
            pltpu.SemaphoreType.DMA,
            pltpu.SemaphoreType.DMA,
        ],
    )
    return f(table, boxes_p)


def kernel(image_shape, boxes, scores, fpn0, fpn1, fpn2, fpn3, fpn4):
    del image_shape, scores
    table = jnp.concatenate(
        [f[0].reshape(-1, C).astype(jnp.float32)
         for f in (fpn0, fpn1, fpn2, fpn3, fpn4)], axis=0)
    b0 = boxes[0].astype(jnp.float32)
    n = b0.shape[0]
    pad = jnp.broadcast_to(jnp.array([0.0, 0.0, 1.0, 1.0], jnp.float32),
                           (NPAD - n, 4))
    boxes_p = jnp.concatenate([b0, pad], axis=0).reshape(-1)
    out = _roi_align(table, boxes_p)
    return jnp.transpose(out, (2, 0, 1, 3)).reshape(1, n, CH, CW, C)


# speedup vs baseline: 49.7925x; 1.0003x over previous
import jax
import jax.numpy as jnp
import numpy as np
from jax import lax
from jax.experimental import pallas as pl
from jax.experimental.pallas import tpu as pltpu
from jax.experimental.pallas import tpu_sc as plsc

CH, CW = 14, 14
C = 256
IMG = 512.0
EPS = 1e-7
SIZES = (64, 32, 16, 8, 4)
BASES = (0, 4096, 5120, 5376, 5440)
NC, NS = 2, 16
NW = NC * NS
N = 1000
NPAD = 1024
BPW = NPAD // NW
GRP = 8
THRESH = tuple((224.0 * (2.0 ** (k - 1) - EPS)) ** 2 for k in (1, 2, 3, 4))
RIMG = 1.0 / IMG
R13 = float(np.float32(1.0) / np.float32(13.0))
RHM1 = tuple(float(np.float32(1.0) / np.float32(s - 1)) for s in SIZES)


def _ifloor(v):
    t = v.astype(jnp.int32)
    return t - jnp.where(t.astype(jnp.float32) > v, jnp.int32(1), jnp.int32(0))


def _fdiv(a, b, rb):
    q = a * rb
    return q + (a - q * b) * rb


_GDN = lax.GatherDimensionNumbers(offset_dims=(), collapsed_slice_dims=(0,),
                                  start_index_map=(0,))


def _vgather(v, idx):
    return lax.gather(v, idx[:, None], _GDN, slice_sizes=(1,),
                      mode=lax.GatherScatterMode.PROMISE_IN_BOUNDS)


def _pack16f(vals):
    lanes = lax.iota(jnp.int32, 16)
    v = jnp.full((16,), 0.0, jnp.float32)
    for k, s in enumerate(vals):
        v = jnp.where(lanes == k, s, v)
    return v


def _box_state(x1, y1, x2, y2):
    wh = (x2 - x1) * (y2 - y1)
    lvl = ((wh >= THRESH[0]).astype(jnp.int32)
           + (wh >= THRESH[1]).astype(jnp.int32)
           + (wh >= THRESH[2]).astype(jnp.int32)
           + (wh >= THRESH[3]).astype(jnp.int32))
    fh = jnp.float32(SIZES[0])
    base = jnp.int32(BASES[0])
    rhm1 = jnp.float32(RHM1[0])
    for k in range(1, 5):
        sel = lvl == k
        fh = jnp.where(sel, jnp.float32(SIZES[k]), fh)
        base = jnp.where(sel, jnp.int32(BASES[k]), base)
        rhm1 = jnp.where(sel, jnp.float32(RHM1[k]), rhm1)
    w = fh.astype(jnp.int32)
    hm1 = fh - 1.0
    y1n = _fdiv(y1 * RIMG * fh, hm1, rhm1)
    x1n = _fdiv(x1 * RIMG * fh, hm1, rhm1)
    y2n = _fdiv(y2 * RIMG * fh - 1.0, hm1, rhm1)
    x2n = _fdiv(x2 * RIMG * fh - 1.0, hm1, rhm1)
    tx_v = _fdiv(lax.iota(jnp.int32, 16).astype(jnp.float32), 13.0, R13)
    xs_v = (x1n + (x2n - x1n) * tx_v) * hm1
    x0t_v = _ifloor(xs_v)
    x0c_v = jnp.minimum(jnp.maximum(x0t_v, 0), w - 1)
    x1c_v = jnp.minimum(jnp.maximum(x0t_v + 1, 0), w - 1)
    packed = _pack16f([hm1, y1n, y2n, x1n, x2n,
                       base.astype(jnp.float32), fh])
    return packed, x0c_v, x1c_v


def _yrow(st, i):
    (base, w, hm1, y1n, y2n, _, _) = st
    ty = _fdiv(lax.convert_element_type(i, jnp.float32), 13.0, R13)
    ys = (y1n + (y2n - y1n) * ty) * hm1
    vy = jnp.where((ys >= 0.0) & (ys <= hm1), 1.0, 0.0)
    y0t = _ifloor(ys)
    ay = ys - y0t.astype(jnp.float32)
    y0c = jnp.minimum(jnp.maximum(y0t, 0), w - 1)
    y1c = jnp.minimum(jnp.maximum(y0t + 1, 0), w - 1)
    return base + y0c * w, base + y1c * w, vy, ay


def _load_state(stf, stx0, stx1, b8):
    vf = stf[b8, :]
    base = vf[5].astype(jnp.int32)
    w = vf[6].astype(jnp.int32)
    st = (base, w, vf[0], vf[1], vf[2], vf[3], vf[4])
    return st, stx0[b8, :], stx1[b8, :]


def _roi_body(table, boxes, out, boxes_v, stf, stx0, stx1, idxa, idxb,
              rowsa, rowsb, oba, obb, semga, semgb, semoa, semob):
    wid = lax.axis_index("s") * NC + lax.axis_index("c")
    box0 = wid * BPW
    pltpu.sync_copy(boxes.at[pl.ds(box0 * 4, BPW * 4)],
                    boxes_v.at[pl.ds(0, BPW * 4)])
    idx2 = (idxa, idxb)
    rows2 = (rowsa, rowsb)
    ob2 = (oba, obb)
    semg2 = (semga, semgb)
    semo2 = (semoa, semob)

    def grp_loop(grp, carry):
        gbase = box0 + grp * GRP

        @pl.when(gbase < N)
        def _():
            qa = boxes_v[pl.ds(grp * 32, 16)]
            qb = boxes_v[pl.ds(grp * 32 + 16, 16)]
            for b8 in range(GRP):
                q = qa if b8 < 4 else qb
                r = 4 * (b8 % 4)
                packed, x0c_v, x1c_v = _box_state(q[r + 0], q[r + 1],
                                                  q[r + 2], q[r + 3])
                stf[b8, :] = packed
                stx0[b8, :] = x0c_v
                stx1[b8, :] = x1c_v

            lanes = lax.iota(jnp.int32, 16)

            def issue(i, b8, par):
                st, x0c_v, x1c_v = _load_state(stf, stx0, stx1, b8)
                r0, r1, _, _ = _yrow(st, i)
                a = r0 + x0c_v
                b = r0 + x1c_v
                c = r1 + x0c_v
                d = r1 + x1c_v
                cl = lambda k: jnp.minimum(jnp.maximum(lanes + k, 0), 15)
                v0 = jnp.where(lanes < 14, a, _vgather(b, cl(-14)))
                v1 = jnp.where(lanes < 12, _vgather(b, cl(2)),
                               _vgather(c, cl(-12)))
                v2 = jnp.where(lanes < 10, _vgather(c, cl(4)),
                               _vgather(d, cl(-10)))
                v3 = _vgather(d, jnp.minimum(lanes + 6, 13))
                idxb_ = idx2[par]
                idxb_[pl.ds(0, 16)] = v0
                idxb_[pl.ds(16, 16)] = v1
                idxb_[pl.ds(32, 16)] = v2
                idxb_[pl.ds(48, 16)] = v3
                pltpu.async_copy(table.at[idxb_.at[pl.ds(0, 56)]],
                                 rows2[par], semg2[par])

            def blend(i, b8, par, ob):
                st, _, _ = _load_state(stf, stx0, stx1, b8)
                (_, _, hm1, _, _, x1n, x2n) = st
                _, _, vy, ay = _yrow(st, i)
                rowb = rows2[par]

                def col_loop(j, c):
                    tx = _fdiv(j.astype(jnp.float32), 13.0, R13)
                    xs = (x1n + (x2n - x1n) * tx) * hm1
                    vx = jnp.where((xs >= 0.0) & (xs <= hm1), 1.0, 0.0)
                    x0t = _ifloor(xs)
                    ax = xs - x0t.astype(jnp.float32)
                    m = vy * vx
                    w00 = (1.0 - ay) * (1.0 - ax) * m
                    w01 = (1.0 - ay) * ax * m
                    w10 = ay * (1.0 - ax) * m
                    w11 = ay * ax * m
                    for cc in range(C // 16):
                        cs = pl.ds(cc * 16, 16)
                        ob[j, b8, cs] = (
                            w00 * rowb[j, cs] + w01 * rowb[j + 14, cs]
                            + w10 * rowb[j + 28, cs]
                            + w11 * rowb[j + 42, cs])
                    return c

                lax.fori_loop(0, CW, col_loop, 0)

            def slot(i, b8, par, obpar):
                nwrap = b8 == GRP - 1
                ni = jnp.where(nwrap, i + 1, i)
                nb8 = jnp.where(nwrap, 0, b8 + 1)

                @pl.when(ni < CH)
                def _():
                    issue(ni, nb8, 1 - par)

                pltpu.make_async_copy(table.at[idx2[par].at[pl.ds(0, 56)]],
                                      rows2[par], semg2[par]).wait()
                blend(i, b8, par, ob2[obpar])

            def half(i, obpar, t):
                @pl.when(t >= 1)
                def _():
                    pltpu.make_async_copy(
                        ob2[obpar],
                        out.at[0, pl.ds(0, CW), pl.ds(gbase, GRP)],
                        semo2[obpar]).wait()

                def b_loop(it, c2):
                    slot(i, it * 2, 0, obpar)
                    slot(i, it * 2 + 1, 1, obpar)
                    return c2

                lax.fori_loop(0, GRP // 2, b_loop, 0)
                pltpu.async_copy(ob2[obpar],
                                 out.at[i, pl.ds(0, CW), pl.ds(gbase, GRP)],
                                 semo2[obpar])

            issue(jnp.int32(0), jnp.int32(0), 0)

            def i_loop(t, carry2):
                half(2 * t, 0, t)
                half(2 * t + 1, 1, t)
                return carry2

            lax.fori_loop(0, CH // 2, i_loop, 0)
            for par in range(2):
                pltpu.make_async_copy(
                    ob2[par], out.at[0, pl.ds(0, CW), pl.ds(gbase, GRP)],
                    semo2[par]).wait()

        return carry

    lax.fori_loop(0, BPW // GRP, grp_loop, 0)


@jax.jit
def _roi_align(table, boxes_p):
    mesh = plsc.VectorSubcoreMesh(core_axis_name="c", subcore_axis_name="s",
                                  num_cores=NC, num_subcores=NS)
    f = pl.kernel(
        _roi_body,
        out_type=jax.ShapeDtypeStruct((CH, CW, N, C), jnp.float32),
        mesh=mesh,
        scratch_types=[
            pltpu.VMEM((BPW * 4 + 16,), jnp.float32),
            pltpu.VMEM((GRP, 16), jnp.float32),
            pltpu.VMEM((GRP, 16), jnp.int32),
            pltpu.VMEM((GRP, 16), jnp.int32),
            pltpu.VMEM((64,), jnp.int32),
            pltpu.VMEM((64,), jnp.int32),
            pltpu.VMEM((56, C), jnp.float32),
            pltpu.VMEM((56, C), jnp.float32),
            pltpu.VMEM((CW, GRP, C), jnp.float32),
            pltpu.VMEM((CW, GRP, C), jnp.float32),
            pltpu.SemaphoreType.DMA,
            pltpu.SemaphoreType.DMA,
            pltpu.SemaphoreType.DMA,
            pltpu.SemaphoreType.DMA,
        ],
    )
    return f(table, boxes_p)


def kernel(image_shape, boxes, scores, fpn0, fpn1, fpn2, fpn3, fpn4):
    del image_shape, scores
    table = jnp.concatenate(
        [f[0].reshape(-1, C).astype(jnp.float32)
         for f in (fpn0, fpn1, fpn2, fpn3, fpn4)], axis=0)
    b0 = boxes[0].astype(jnp.float32)
    n = b0.shape[0]
    pad = jnp.broadcast_to(jnp.array([0.0, 0.0, 1.0, 1.0], jnp.float32),
                           (NPAD - n, 4))
    boxes_p = jnp.concatenate([b0, pad], axis=0).reshape(-1)
    out = _roi_align(table, boxes_p)
    return jnp.transpose(out, (2, 0, 1, 3)).reshape(1, n, CH, CW, C)
